# baseline (speedup 1.0000x reference)
import functools

import jax
import jax.numpy as jnp
from jax import lax
from jax.experimental import pallas as pl
from jax.experimental.pallas import tpu as pltpu
from jax.experimental.pallas import tpu_sc as plsc

_NC = 2
_NS = 16
_NW = _NC * _NS


def _gather_body(per_w, dw, idx_hbm, tbl_hbm, out_hbm, idx_v, bund_v, rows_v, sem):
    bw = 4 * dw
    wid = lax.axis_index("s") * _NC + lax.axis_index("c")
    base = wid * per_w
    pltpu.sync_copy(idx_hbm.at[pl.ds(base, per_w)], idx_v)

    @pl.loop(0, per_w, step=16)
    def _fire(g):
        v = idx_v[pl.ds(g, 16)]
        for j in range(16):
            pltpu.make_async_copy(
                tbl_hbm.at[pl.ds((v[j] // 4) * bw, bw)],
                bund_v.at[pl.ds((g + j) * bw, bw)],
                sem,
            ).start()

    pltpu.make_async_copy(tbl_hbm.at[pl.ds(0, per_w * bw)], bund_v, sem).wait()

    @pl.loop(0, per_w, step=16)
    def _select(g):
        v = idx_v[pl.ds(g, 16)]
        for j in range(16):
            src = (g + j) * bw + (v[j] % 4) * dw
            dst = (g + j) * dw
            rows_v[pl.ds(dst, 16)] = bund_v[pl.ds(src, 16)]
            rows_v[pl.ds(dst + 16, 16)] = bund_v[pl.ds(src + 16, 16)]

    pltpu.sync_copy(rows_v, out_hbm.at[pl.ds(base * dw, per_w * dw)])


def _sc_gather(idx, tbl_words, dw):
    m = idx.shape[0]
    per_w = m // _NW
    mesh = plsc.VectorSubcoreMesh(core_axis_name="c", subcore_axis_name="s")
    kern = functools.partial(
        pl.kernel,
        mesh=mesh,
        out_type=jax.ShapeDtypeStruct((m * dw,), jnp.int32),
        scratch_types=[
            pltpu.VMEM((per_w,), jnp.int32),
            pltpu.VMEM((per_w * 4 * dw,), jnp.int32),
            pltpu.VMEM((per_w * dw,), jnp.int32),
            pltpu.SemaphoreType.DMA,
        ],
    )(functools.partial(_gather_body, per_w, dw))
    return kern(idx, tbl_words)


def _mlp_body(e_ref, w1_ref, b1_ref, w2_ref, b2_ref, o_ref):
    h = jnp.dot(e_ref[...], w1_ref[...], preferred_element_type=jnp.float32)
    h = jax.nn.gelu(h + b1_ref[...])
    o_ref[...] = (
        jnp.dot(h.astype(jnp.bfloat16), w2_ref[...], preferred_element_type=jnp.float32)
        + b2_ref[...]
    )


def _tc_mlp(embeds, W1, b1, W2, b2, block_m=1024):
    m, k = embeds.shape
    h = W1.shape[1]
    n = W2.shape[1]
    grid = (m // block_m,)
    return pl.pallas_call(
        _mlp_body,
        grid=grid,
        in_specs=[
            pl.BlockSpec((block_m, k), lambda i: (i, 0)),
            pl.BlockSpec((k, h), lambda i: (0, 0)),
            pl.BlockSpec((1, h), lambda i: (0, 0)),
            pl.BlockSpec((h, n), lambda i: (0, 0)),
            pl.BlockSpec((1, n), lambda i: (0, 0)),
        ],
        out_specs=pl.BlockSpec((block_m, n), lambda i: (i, 0)),
        out_shape=jax.ShapeDtypeStruct((m, n), jnp.float32),
    )(embeds, W1, b1, W2, b2)


def kernel(token_ids, table, W1, b1, W2, b2):
    b, s = token_ids.shape
    v, d = table.shape
    m = b * s
    dw = d // 2
    n_out = W2.shape[1]
    tbl_words = lax.bitcast_convert_type(
        table.astype(jnp.bfloat16).reshape(-1, 2), jnp.int32
    )
    idx = token_ids.T.reshape(-1)
    rows = _sc_gather(idx, tbl_words, dw)
    embeds = lax.bitcast_convert_type(rows.reshape(m, dw), jnp.bfloat16).reshape(m, d)
    out = _tc_mlp(
        embeds,
        W1.astype(jnp.bfloat16),
        b1.reshape(1, -1),
        W2.astype(jnp.bfloat16),
        b2.reshape(1, -1),
    )
    return out.reshape(s, b, n_out).transpose(1, 0, 2)


# speedup vs baseline: 11.2205x; 11.2205x over previous
import functools

import jax
import jax.numpy as jnp
from jax import lax
from jax.experimental import pallas as pl
from jax.experimental.pallas import tpu as pltpu
from jax.experimental.pallas import tpu_sc as plsc

_NC = 2
_NS = 16
_NW = _NC * _NS


_CH = 160


def _gather_body(per_w, dw, idx_hbm, tbl_hbm, out_hbm, idx_v, bund_v, rows_v, s0, s1):
    wid = lax.axis_index("s") * _NC + lax.axis_index("c")
    base = wid * per_w
    sems = (s0, s1)
    n_ch = per_w // _CH
    pltpu.sync_copy(idx_hbm.at[pl.ds(base, per_w)], idx_v)

    def fire(c):
        buf = c % 2

        @pl.loop(0, _CH, step=16)
        def _fire(g):
            v = idx_v[pl.ds(c * _CH + g, 16)]
            for j in range(16):
                pltpu.make_async_copy(
                    tbl_hbm.at[pl.ds(v[j] // 4, 1)],
                    bund_v.at[buf, pl.ds(g + j, 1)],
                    sems[buf],
                ).start()

    fire(0)
    for c in range(n_ch):
        buf = c % 2
        if c + 1 < n_ch:
            fire(c + 1)
        pltpu.make_async_copy(tbl_hbm.at[pl.ds(0, _CH)], bund_v.at[buf], sems[buf]).wait()

        @pl.loop(0, _CH, step=16)
        def _select(g):
            v = idx_v[pl.ds(c * _CH + g, 16)]
            for j in range(16):
                off = (v[j] % 4) * dw
                t = c * _CH + g + j
                rows_v[t, pl.ds(0, 16)] = bund_v[buf, g + j, pl.ds(off, 16)]
                rows_v[t, pl.ds(16, 16)] = bund_v[buf, g + j, pl.ds(off + 16, 16)]

    pltpu.sync_copy(rows_v, out_hbm.at[pl.ds(base, per_w)])


def _sc_gather(idx, tbl, dw):
    m = idx.shape[0]
    per_w = m // _NW
    mesh = plsc.VectorSubcoreMesh(core_axis_name="c", subcore_axis_name="s")
    kern = functools.partial(
        pl.kernel,
        mesh=mesh,
        out_type=jax.ShapeDtypeStruct((m, dw), jnp.uint32),
        scratch_types=[
            pltpu.VMEM((per_w,), jnp.int32),
            pltpu.VMEM((2, _CH, 4 * dw), jnp.uint32),
            pltpu.VMEM((per_w, dw), jnp.uint32),
            pltpu.SemaphoreType.DMA,
            pltpu.SemaphoreType.DMA,
        ],
    )(functools.partial(_gather_body, per_w, dw))
    return kern(idx, tbl)


def _mlp_body(e_ref, w1_ref, b1_ref, w2_ref, b2_ref, o_ref):
    h = jnp.dot(e_ref[...], w1_ref[...], preferred_element_type=jnp.float32)
    h = jax.nn.gelu(h + b1_ref[...])
    o_ref[...] = (
        jnp.dot(h.astype(jnp.bfloat16), w2_ref[...], preferred_element_type=jnp.float32)
        + b2_ref[...]
    )


def _tc_mlp(embeds, W1, b1, W2, b2, block_m=1024):
    m, k = embeds.shape
    h = W1.shape[1]
    n = W2.shape[1]
    grid = (m // block_m,)
    return pl.pallas_call(
        _mlp_body,
        grid=grid,
        in_specs=[
            pl.BlockSpec((block_m, k), lambda i: (i, 0)),
            pl.BlockSpec((k, h), lambda i: (0, 0)),
            pl.BlockSpec((1, h), lambda i: (0, 0)),
            pl.BlockSpec((h, n), lambda i: (0, 0)),
            pl.BlockSpec((1, n), lambda i: (0, 0)),
        ],
        out_specs=pl.BlockSpec((block_m, n), lambda i: (i, 0)),
        out_shape=jax.ShapeDtypeStruct((m, n), jnp.float32),
    )(embeds, W1, b1, W2, b2)


def kernel(token_ids, table, W1, b1, W2, b2):
    b, s = token_ids.shape
    v, d = table.shape
    m = b * s
    dw = d // 2
    n_out = W2.shape[1]
    tb = lax.bitcast_convert_type(table.astype(jnp.bfloat16), jnp.uint16)
    lo = tb[:, 0::2].astype(jnp.uint32)
    hi = tb[:, 1::2].astype(jnp.uint32)
    words = lo | (hi << 16)
    tbl = words.reshape(v // 4, 4 * dw)
    idx = token_ids.T.reshape(-1)
    rows = _sc_gather(idx, tbl, dw)
    embeds = lax.bitcast_convert_type(rows, jnp.bfloat16).reshape(m, d)
    out = _tc_mlp(
        embeds,
        W1.astype(jnp.bfloat16),
        b1.reshape(1, -1),
        W2.astype(jnp.bfloat16),
        b2.reshape(1, -1),
    )
    return out.reshape(s, b, n_out).transpose(1, 0, 2)


# speedup vs baseline: 33.1523x; 2.9546x over previous
import functools

import jax
import jax.numpy as jnp
from jax import lax
from jax.experimental import pallas as pl
from jax.experimental.pallas import tpu as pltpu
from jax.experimental.pallas import tpu_sc as plsc

_NC = 2
_NS = 16
_NW = _NC * _NS
_CH = 160


def _gather_body(per_w, d, idx_hbm, tbl_hbm, out_hbm, idx_v, bund_v, rows_v, s0, s1):
    wid = lax.axis_index("s") * _NC + lax.axis_index("c")
    base = wid * per_w
    sems = (s0, s1)
    n_ch = per_w // _CH
    pltpu.sync_copy(idx_hbm.at[pl.ds(base, per_w)], idx_v)

    def fire(c):
        buf = c % 2

        @pl.loop(0, _CH, step=16)
        def _fire(g):
            v = idx_v[pl.ds(c * _CH + g, 16)]
            for j in range(16):
                pltpu.make_async_copy(
                    tbl_hbm.at[pl.ds(v[j] // 2, 1)],
                    bund_v.at[buf, pl.ds(g + j, 1)],
                    sems[buf],
                ).start()

    fire(0)
    for c in range(n_ch):
        buf = c % 2
        if c + 1 < n_ch:
            fire(c + 1)
        pltpu.make_async_copy(tbl_hbm.at[pl.ds(0, _CH)], bund_v.at[buf], sems[buf]).wait()

        @pl.loop(0, _CH, step=16)
        def _select(g):
            v = idx_v[pl.ds(c * _CH + g, 16)]
            for j in range(16):
                off = (v[j] % 2) * d
                t = c * _CH + g + j
                for q in range(0, d, 16):
                    rows_v[t, pl.ds(q, 16)] = bund_v[buf, g + j, pl.ds(off + q, 16)]

    pltpu.sync_copy(rows_v, out_hbm.at[pl.ds(base, per_w)])


def _sc_gather(idx, tbl, d):
    m = idx.shape[0]
    per_w = m // _NW
    mesh = plsc.VectorSubcoreMesh(core_axis_name="c", subcore_axis_name="s")
    kern = functools.partial(
        pl.kernel,
        mesh=mesh,
        out_type=jax.ShapeDtypeStruct((m, d), tbl.dtype),
        scratch_types=[
            pltpu.VMEM((per_w,), jnp.int32),
            pltpu.VMEM((2, _CH, 2 * d), tbl.dtype),
            pltpu.VMEM((per_w, d), tbl.dtype),
            pltpu.SemaphoreType.DMA,
            pltpu.SemaphoreType.DMA,
        ],
    )(functools.partial(_gather_body, per_w, d))
    return kern(idx, tbl)


def _mlp_body(e_ref, w1_ref, b1_ref, w2_ref, b2_ref, o_ref):
    h = jnp.dot(e_ref[...], w1_ref[...], preferred_element_type=jnp.float32)
    h = jax.nn.gelu(h + b1_ref[...])
    o_ref[...] = jnp.dot(h, w2_ref[...], preferred_element_type=jnp.float32) + b2_ref[...]


def _tc_mlp(embeds, W1, b1, W2, b2, block_m=1024):
    m, k = embeds.shape
    h = W1.shape[1]
    n = W2.shape[1]
    grid = (m // block_m,)
    return pl.pallas_call(
        _mlp_body,
        grid=grid,
        in_specs=[
            pl.BlockSpec((block_m, k), lambda i: (i, 0)),
            pl.BlockSpec((k, h), lambda i: (0, 0)),
            pl.BlockSpec((1, h), lambda i: (0, 0)),
            pl.BlockSpec((h, n), lambda i: (0, 0)),
            pl.BlockSpec((1, n), lambda i: (0, 0)),
        ],
        out_specs=pl.BlockSpec((block_m, n), lambda i: (i, 0)),
        out_shape=jax.ShapeDtypeStruct((m, n), jnp.float32),
    )(embeds, W1, b1, W2, b2)


def kernel(token_ids, table, W1, b1, W2, b2):
    b, s = token_ids.shape
    v, d = table.shape
    n_out = W2.shape[1]
    tbl = table.reshape(v // 2, 2 * d)
    idx = token_ids.T.reshape(-1)
    embeds = _sc_gather(idx, tbl, d)
    out = _tc_mlp(embeds, W1, b1.reshape(1, -1), W2, b2.reshape(1, -1))
    return out.reshape(s, b, n_out).transpose(1, 0, 2)


# speedup vs baseline: 42.2015x; 1.2730x over previous
import functools

import jax
import jax.numpy as jnp
from jax import lax
from jax.experimental import pallas as pl
from jax.experimental.pallas import tpu as pltpu
from jax.experimental.pallas import tpu_sc as plsc

_NC = 2
_NS = 16
_NW = _NC * _NS


def _tr_body(in_ref, o_ref):
    o_ref[...] = in_ref[...].T


def _tc_transpose(tableT, block_l=2048):
    d, v = tableT.shape
    grid = (pl.cdiv(v, block_l),)
    return pl.pallas_call(
        _tr_body,
        grid=grid,
        in_specs=[pl.BlockSpec((d, block_l), lambda i: (0, i))],
        out_specs=pl.BlockSpec((block_l, d), lambda i: (i, 0)),
        out_shape=jax.ShapeDtypeStruct((v, d), tableT.dtype),
    )(tableT)


def _gather_body(per_w, idx_hbm, table_hbm, out_hbm, idx_v, rows_v, sem):
    wid = lax.axis_index("s") * _NC + lax.axis_index("c")
    base = wid * per_w
    pltpu.sync_copy(idx_hbm.at[pl.ds(base, per_w)], idx_v)

    @pl.loop(0, per_w, step=16)
    def _fire(g):
        v = idx_v[pl.ds(g, 16)]
        for j in range(16):
            pltpu.make_async_copy(
                table_hbm.at[pl.ds(v[j], 1)], rows_v.at[pl.ds(g + j, 1)], sem
            ).start()

    pltpu.make_async_copy(out_hbm.at[pl.ds(base, per_w)], rows_v, sem).wait()
    pltpu.sync_copy(rows_v, out_hbm.at[pl.ds(base, per_w)])


def _sc_gather(idx, table):
    m = idx.shape[0]
    d = table.shape[1]
    per_w = m // _NW
    mesh = plsc.VectorSubcoreMesh(core_axis_name="c", subcore_axis_name="s")
    kern = functools.partial(
        pl.kernel,
        mesh=mesh,
        out_type=jax.ShapeDtypeStruct((m, d), table.dtype),
        scratch_types=[
            pltpu.VMEM((per_w,), jnp.int32),
            pltpu.VMEM((per_w, d), table.dtype),
            pltpu.SemaphoreType.DMA,
        ],
    )(functools.partial(_gather_body, per_w))
    return kern(idx, table)


def _mlp_body(e_ref, w1_ref, b1_ref, w2_ref, b2_ref, o_ref):
    h = jnp.dot(e_ref[...], w1_ref[...], preferred_element_type=jnp.float32)
    h = jax.nn.gelu(h + b1_ref[...])
    o_ref[...] = jnp.dot(h, w2_ref[...], preferred_element_type=jnp.float32) + b2_ref[...]


def _tc_mlp(embeds, W1, b1, W2, b2, block_m=1024):
    m, k = embeds.shape
    h = W1.shape[1]
    n = W2.shape[1]
    grid = (m // block_m,)
    return pl.pallas_call(
        _mlp_body,
        grid=grid,
        in_specs=[
            pl.BlockSpec((block_m, k), lambda i: (i, 0)),
            pl.BlockSpec((k, h), lambda i: (0, 0)),
            pl.BlockSpec((1, h), lambda i: (0, 0)),
            pl.BlockSpec((h, n), lambda i: (0, 0)),
            pl.BlockSpec((1, n), lambda i: (0, 0)),
        ],
        out_specs=pl.BlockSpec((block_m, n), lambda i: (i, 0)),
        out_shape=jax.ShapeDtypeStruct((m, n), jnp.float32),
    )(embeds, W1, b1, W2, b2)


def kernel(token_ids, table, W1, b1, W2, b2):
    b, s = token_ids.shape
    n_out = W2.shape[1]
    tbl_rm = _tc_transpose(table.T)
    idx = token_ids.T.reshape(-1)
    embeds = _sc_gather(idx, tbl_rm)
    out = _tc_mlp(embeds, W1, b1.reshape(1, -1), W2, b2.reshape(1, -1))
    return out.reshape(s, b, n_out).transpose(1, 0, 2)


# speedup vs baseline: 68.0840x; 1.6133x over previous
import functools

import jax
import jax.numpy as jnp
from jax import lax
from jax.experimental import pallas as pl
from jax.experimental.pallas import tpu as pltpu
from jax.experimental.pallas import tpu_sc as plsc

_NC = 2
_NS = 16
_NW = _NC * _NS


def _tr_body(in_ref, o_ref):
    o_ref[...] = in_ref[...].T


def _tc_transpose(tableT, block_l=8192):
    d, v = tableT.shape
    grid = (pl.cdiv(v, block_l),)
    return pl.pallas_call(
        _tr_body,
        grid=grid,
        in_specs=[pl.BlockSpec((d, block_l), lambda i: (0, i))],
        out_specs=pl.BlockSpec((block_l, d), lambda i: (i, 0)),
        out_shape=jax.ShapeDtypeStruct((v, d), tableT.dtype),
    )(tableT)


def _gather_body(per_w, idx_hbm, table_hbm, out_hbm, idx_v, rows_v, sem):
    wid = lax.axis_index("s") * _NC + lax.axis_index("c")
    base = wid * per_w
    pltpu.sync_copy(idx_hbm.at[pl.ds(base, per_w)], idx_v)

    @pl.loop(0, per_w, step=16)
    def _fire(g):
        v = idx_v[pl.ds(g, 16)]
        for j in range(16):
            pltpu.make_async_copy(
                table_hbm.at[pl.ds(v[j], 1)], rows_v.at[pl.ds(g + j, 1)], sem
            ).start()

    pltpu.make_async_copy(out_hbm.at[pl.ds(base, per_w)], rows_v, sem).wait()
    pltpu.sync_copy(rows_v, out_hbm.at[pl.ds(base, per_w)])


def _sc_gather(idx, table):
    m = idx.shape[0]
    d = table.shape[1]
    per_w = m // _NW
    mesh = plsc.VectorSubcoreMesh(core_axis_name="c", subcore_axis_name="s")
    kern = functools.partial(
        pl.kernel,
        mesh=mesh,
        out_type=jax.ShapeDtypeStruct((m, d), table.dtype),
        scratch_types=[
            pltpu.VMEM((per_w,), jnp.int32),
            pltpu.VMEM((per_w, d), table.dtype),
            pltpu.SemaphoreType.DMA,
        ],
    )(functools.partial(_gather_body, per_w))
    return kern(idx, table)


def _mlp_body(e_ref, w1_ref, b1_ref, w2_ref, b2_ref, o_ref):
    h = jnp.dot(e_ref[...], w1_ref[...], preferred_element_type=jnp.float32)
    h = jax.nn.gelu(h + b1_ref[...])
    o_ref[...] = jnp.dot(h, w2_ref[...], preferred_element_type=jnp.float32) + b2_ref[...]


def _tc_mlp(embeds, W1, b1, W2, b2, block_m=1024):
    m, k = embeds.shape
    h = W1.shape[1]
    n = W2.shape[1]
    grid = (m // block_m,)
    return pl.pallas_call(
        _mlp_body,
        grid=grid,
        in_specs=[
            pl.BlockSpec((block_m, k), lambda i: (i, 0)),
            pl.BlockSpec((k, h), lambda i: (0, 0)),
            pl.BlockSpec((1, h), lambda i: (0, 0)),
            pl.BlockSpec((h, n), lambda i: (0, 0)),
            pl.BlockSpec((1, n), lambda i: (0, 0)),
        ],
        out_specs=pl.BlockSpec((block_m, n), lambda i: (i, 0)),
        out_shape=jax.ShapeDtypeStruct((m, n), jnp.float32),
    )(embeds, W1, b1, W2, b2)


def kernel(token_ids, table, W1, b1, W2, b2):
    b, s = token_ids.shape
    n_out = W2.shape[1]
    tbl_rm = _tc_transpose(table.T)
    idx = token_ids.T.reshape(-1)
    embeds = _sc_gather(idx, tbl_rm)
    out = _tc_mlp(embeds, W1, b1.reshape(1, -1), W2, b2.reshape(1, -1))
    return out.reshape(s, b, n_out).transpose(1, 0, 2)


# speedup vs baseline: 73.2492x; 1.0759x over previous
import functools

import jax
import jax.numpy as jnp
from jax import lax
from jax.experimental import pallas as pl
from jax.experimental.pallas import tpu as pltpu
from jax.experimental.pallas import tpu_sc as plsc

_NC = 2
_NS = 16
_NW = _NC * _NS


def _tr_body(in_ref, o_ref):
    o_ref[...] = in_ref[...].T


def _tc_transpose(tableT, block_l=32768):
    d, v = tableT.shape
    grid = (pl.cdiv(v, block_l),)
    return pl.pallas_call(
        _tr_body,
        grid=grid,
        in_specs=[pl.BlockSpec((d, block_l), lambda i: (0, i))],
        out_specs=pl.BlockSpec((block_l, d), lambda i: (i, 0)),
        out_shape=jax.ShapeDtypeStruct((v, d), tableT.dtype),
    )(tableT)


def _gather_body(per_w, idx_hbm, table_hbm, out_hbm, idx_v, rows_v, sem):
    wid = lax.axis_index("s") * _NC + lax.axis_index("c")
    base = wid * per_w
    pltpu.sync_copy(idx_hbm.at[pl.ds(base, per_w)], idx_v)

    @pl.loop(0, per_w, step=16)
    def _fire(g):
        v = idx_v[pl.ds(g, 16)]
        for j in range(16):
            pltpu.make_async_copy(
                table_hbm.at[pl.ds(v[j], 1)], rows_v.at[pl.ds(g + j, 1)], sem
            ).start()

    pltpu.make_async_copy(out_hbm.at[pl.ds(base, per_w)], rows_v, sem).wait()
    pltpu.sync_copy(rows_v, out_hbm.at[pl.ds(base, per_w)])


def _sc_gather(idx, table):
    m = idx.shape[0]
    d = table.shape[1]
    per_w = m // _NW
    mesh = plsc.VectorSubcoreMesh(core_axis_name="c", subcore_axis_name="s")
    kern = functools.partial(
        pl.kernel,
        mesh=mesh,
        out_type=jax.ShapeDtypeStruct((m, d), table.dtype),
        scratch_types=[
            pltpu.VMEM((per_w,), jnp.int32),
            pltpu.VMEM((per_w, d), table.dtype),
            pltpu.SemaphoreType.DMA,
        ],
    )(functools.partial(_gather_body, per_w))
    return kern(idx, table)


def _mlp_body(e_ref, w1_ref, b1_ref, w2_ref, b2_ref, o_ref):
    h = jnp.dot(e_ref[...], w1_ref[...], preferred_element_type=jnp.float32)
    h = jax.nn.gelu(h + b1_ref[...])
    o_ref[...] = jnp.dot(h, w2_ref[...], preferred_element_type=jnp.float32) + b2_ref[...]


def _tc_mlp(embeds, W1, b1, W2, b2, block_m=1024):
    m, k = embeds.shape
    h = W1.shape[1]
    n = W2.shape[1]
    grid = (m // block_m,)
    return pl.pallas_call(
        _mlp_body,
        grid=grid,
        in_specs=[
            pl.BlockSpec((block_m, k), lambda i: (i, 0)),
            pl.BlockSpec((k, h), lambda i: (0, 0)),
            pl.BlockSpec((1, h), lambda i: (0, 0)),
            pl.BlockSpec((h, n), lambda i: (0, 0)),
            pl.BlockSpec((1, n), lambda i: (0, 0)),
        ],
        out_specs=pl.BlockSpec((block_m, n), lambda i: (i, 0)),
        out_shape=jax.ShapeDtypeStruct((m, n), jnp.float32),
    )(embeds, W1, b1, W2, b2)


def kernel(token_ids, table, W1, b1, W2, b2):
    b, s = token_ids.shape
    n_out = W2.shape[1]
    tbl_rm = _tc_transpose(table.T)
    idx = token_ids.T.reshape(-1)
    embeds = _sc_gather(idx, tbl_rm)
    out = _tc_mlp(embeds, W1, b1.reshape(1, -1), W2, b2.reshape(1, -1))
    return out.reshape(s, b, n_out).transpose(1, 0, 2)
